# baseline (speedup 1.0000x reference)
import jax
import jax.numpy as jnp
from jax import lax
from jax.experimental import pallas as pl
from jax.experimental.pallas import tpu as pltpu
from jax.experimental.pallas import tpu_sc as plsc

_D = 128
_N = 10000
_E = 320000
_NT = 16
_EPT = _E // _NT
_W = 16
_ROWS = 1280
_PADR = _ROWS - _EPT // _W
_NBLK = 20
_BR = _ROWS // _NBLK
_NB = 4
_SPAD = 10240
_STRIPE = _SPAD // _NT
_WROWS = _ROWS * _W // 128
_WBR = _WROWS // _NBLK

_PREC = lax.Precision.HIGHEST


def _mlp_body(x_ref, w1_ref, b1_ref, w2_ref, xw_ref, b2_ref, o_ref):
    x = x_ref[...]
    h = jnp.dot(x, w1_ref[...], preferred_element_type=jnp.float32,
                precision=_PREC)
    h = jnp.maximum(h + b1_ref[...], 0.0)
    y = jnp.dot(h, w2_ref[...], preferred_element_type=jnp.float32,
                precision=_PREC)
    y = y + jnp.dot(x, xw_ref[...], preferred_element_type=jnp.float32,
                    precision=_PREC)
    o_ref[...] = y + b2_ref[...]


def _mlp_call(x, w1, b1, w2, xw, b2):
    n, k = x.shape
    f = w2.shape[1]
    br = 1000
    return pl.pallas_call(
        _mlp_body,
        grid=(n // br,),
        in_specs=[
            pl.BlockSpec((br, k), lambda i: (i, 0)),
            pl.BlockSpec((k, k), lambda i: (0, 0)),
            pl.BlockSpec((1, k), lambda i: (0, 0)),
            pl.BlockSpec((k, f), lambda i: (0, 0)),
            pl.BlockSpec((k, f), lambda i: (0, 0)),
            pl.BlockSpec((1, f), lambda i: (0, 0)),
        ],
        out_specs=pl.BlockSpec((br, f), lambda i: (i, 0)),
        out_shape=jax.ShapeDtypeStruct((n, f), jnp.float32),
    )(x, w1, b1.reshape(1, -1), w2, xw, b2.reshape(1, -1))


def _mm_body(x_ref, w_ref, b_ref, o_ref):
    o_ref[...] = jnp.dot(
        x_ref[...], w_ref[...], preferred_element_type=jnp.float32,
        precision=_PREC) + b_ref[...]


def _mm_call(x, w, b):
    n, k = x.shape
    f = w.shape[1]
    br = 1000
    return pl.pallas_call(
        _mm_body,
        grid=(n // br,),
        in_specs=[
            pl.BlockSpec((br, k), lambda i: (i, 0)),
            pl.BlockSpec((k, f), lambda i: (0, 0)),
            pl.BlockSpec((1, f), lambda i: (0, 0)),
        ],
        out_specs=pl.BlockSpec((br, f), lambda i: (i, 0)),
        out_shape=jax.ShapeDtypeStruct((n, f), jnp.float32),
    )(x, w, b.reshape(1, -1))


def _edge_body(t_hbm, p_hbm, q_hbm, ei_hbm, eiw_hbm, out_hbm,
               p_v, q_v, s_v, segw_b, qiw_b, e_blk, zs_v,
               seg_b, qi_b, src_b, rows0, rows1, rows2, rows3, w_v,
               sp_out, sp_s,
               gsem0, gsem1, gsem2, gsem3, ssem0, ssem1, ssem2, ssem3,
               sem_i, sem_s):
    rows_a = rows0
    d = lax.axis_index("c")
    t = lax.axis_index("s")

    z16 = jnp.zeros((16,), jnp.float32)

    pltpu.sync_copy(p_hbm.at[pl.ds(d * _SPAD, _SPAD)], p_v)
    pltpu.sync_copy(q_hbm.at[pl.ds(d * _SPAD, _SPAD)], q_v)

    for k in range(_STRIPE // 16):
        zs_v[pl.ds(k * 16, 16)] = z16
    for r in range(_W):
        for c8 in range(8):
            rows_a[r, pl.ds(c8 * 16, 16)] = z16
    pltpu.sync_copy(zs_v, sp_s.at[pl.ds(t * _STRIPE, _STRIPE)])

    def _zero_out(i, _):
        pltpu.sync_copy(rows_a, sp_out.at[pl.ds(t * _STRIPE + i * _W, _W)])
        return 0
    lax.fori_loop(0, _STRIPE // _W, _zero_out, 0)

    neg = jnp.full((16,), -3.4e38, jnp.float32)

    def _mx(table):
        def step(i, m):
            return jnp.maximum(m, table[pl.ds(i * 16, 16)])
        mv = lax.fori_loop(0, _SPAD // 16, step, neg)
        w_v[pl.ds(0, 16)] = mv
        m = neg
        for k in range(16):
            m = jnp.maximum(
                m, plsc.load_gather(w_v, [jnp.full((16,), k, jnp.int32)]))
        return m

    m_tot = _mx(p_v) + _mx(q_v)
    m_vec = jnp.maximum(m_tot, 0.2 * m_tot)

    def _exp_logit(ci, qi):
        pg = plsc.load_gather(p_v, [ci])
        qg = plsc.load_gather(q_v, [qi])
        x = pg + qg
        xl = jnp.maximum(x, 0.2 * x)
        return jnp.exp(xl - m_vec)

    plsc.subcore_barrier()

    def _p1_block(b, _):
        c1 = pltpu.async_copy(eiw_hbm.at[0, t, pl.ds(b * _WBR, _WBR)],
                              segw_b, sem_i)
        c2 = pltpu.async_copy(eiw_hbm.at[1, t, pl.ds(b * _WBR, _WBR)],
                              qiw_b, sem_i)
        c1.wait()
        c2.wait()

        for j in range(_WBR):
            for k in range(8):
                ci = segw_b[j, pl.ds(k * 16, 16)]
                e_blk[j, pl.ds(k * 16, 16)] = _exp_logit(
                    ci, qiw_b[j, pl.ds(k * 16, 16)])
        for j in range(_WBR):
            pltpu.async_copy(e_blk.at[j], sp_s.at[segw_b.at[j]], sem_s,
                             add=True)
        for j in range(_WBR):
            pltpu.make_async_copy(
                e_blk.at[0], sp_s.at[segw_b.at[0]], sem_s).wait()
        return 0
    lax.fori_loop(0, _NBLK, _p1_block, 0)

    plsc.subcore_barrier()
    pltpu.sync_copy(sp_s, s_v)

    t_view = t_hbm.at[d]
    bufs = ((rows0, gsem0, ssem0), (rows1, gsem1, ssem1),
            (rows2, gsem2, ssem2), (rows3, gsem3, ssem3))

    def _scatter(buf, ssem, jj):
        @pl.when(d == 0)
        def _():
            pltpu.async_copy(buf, sp_out.at[seg_b.at[jj]], ssem, add=True)

        @pl.when(d == 1)
        def _():
            pltpu.async_copy(buf, sp_out.at[qi_b.at[jj]], ssem, add=True)

    def _p2_block(b, _):
        c1 = pltpu.async_copy(ei_hbm.at[0, t, pl.ds(b * _BR, _BR)], seg_b,
                              sem_i)
        c2 = pltpu.async_copy(ei_hbm.at[1, t, pl.ds(b * _BR, _BR)], qi_b,
                              sem_i)
        c3 = pltpu.async_copy(ei_hbm.at[2 + d, t, pl.ds(b * _BR, _BR)],
                              src_b, sem_i)
        c1.wait()
        c2.wait()
        c3.wait()

        for k in range(_NB - 1):
            pltpu.async_copy(t_view.at[src_b.at[k]], bufs[k][0], bufs[k][1])

        def outer(jo, _):
            j0 = jo * _NB
            for bb in range(_NB):
                buf, gsem, ssem = bufs[bb]
                nbuf, ngsem, nssem = bufs[(bb + _NB - 1) % _NB]
                jj = j0 + bb

                @pl.when(jj + _NB - 1 < _BR)
                def _():
                    @pl.when(jj >= 1)
                    def _():
                        pltpu.make_async_copy(
                            nbuf, sp_out.at[seg_b.at[0]], nssem).wait()
                    pltpu.async_copy(
                        t_view.at[src_b.at[jj + _NB - 1]], nbuf, ngsem)

                pltpu.make_async_copy(
                    t_view.at[src_b.at[jj]], buf, gsem).wait()

                ci = seg_b[jj, :]
                e = _exp_logit(ci, qi_b[jj, :])
                sg = plsc.load_gather(s_v, [ci])
                w_v[...] = e / (sg + 1e-16)

                def scale_row(r, _):
                    spl = plsc.load_gather(
                        w_v, [jnp.full((16,), r, jnp.int32)])
                    for c8 in range(8):
                        buf[r, pl.ds(c8 * 16, 16)] = (
                            buf[r, pl.ds(c8 * 16, 16)] * spl)
                    return 0
                lax.fori_loop(0, _W, scale_row, 0)

                _scatter(buf, ssem, jj)
            return 0
        lax.fori_loop(0, _BR // _NB, outer, 0)

        for k in range(_NB):
            pltpu.make_async_copy(
                bufs[k][0], sp_out.at[seg_b.at[0]], bufs[k][2]).wait()
        return 0
    lax.fori_loop(0, _NBLK, _p2_block, 0)

    plsc.subcore_barrier()
    pltpu.sync_copy(sp_out.at[pl.ds(t * _STRIPE, _STRIPE)],
                    out_hbm.at[d, pl.ds(t * _STRIPE, _STRIPE)])


def _edge_aggregate(t_tables, p_tables, q_tables, ei):
    mesh = plsc.VectorSubcoreMesh(core_axis_name="c", subcore_axis_name="s")
    call = pl.kernel(
        _edge_body,
        out_type=jax.ShapeDtypeStruct((2, _SPAD, _D), jnp.float32),
        mesh=mesh,
        compiler_params=pltpu.CompilerParams(
            needs_layout_passes=False, use_tc_tiling_on_sc=False),
        scratch_types=[
            pltpu.VMEM((_SPAD,), jnp.float32),
            pltpu.VMEM((_SPAD,), jnp.float32),
            pltpu.VMEM((_SPAD,), jnp.float32),
            pltpu.VMEM((_WBR, 128), jnp.int32),
            pltpu.VMEM((_WBR, 128), jnp.int32),
            pltpu.VMEM((_WBR, 128), jnp.float32),
            pltpu.VMEM((_STRIPE,), jnp.float32),
            pltpu.VMEM((_BR, _W), jnp.int32),
            pltpu.VMEM((_BR, _W), jnp.int32),
            pltpu.VMEM((_BR, _W), jnp.int32),
            pltpu.VMEM((_W, _D), jnp.float32),
            pltpu.VMEM((_W, _D), jnp.float32),
            pltpu.VMEM((_W, _D), jnp.float32),
            pltpu.VMEM((_W, _D), jnp.float32),
            pltpu.VMEM((_W,), jnp.float32),
            pltpu.VMEM_SHARED((_SPAD, _D), jnp.float32),
            pltpu.VMEM_SHARED((_SPAD,), jnp.float32),
            pltpu.SemaphoreType.DMA,
            pltpu.SemaphoreType.DMA,
            pltpu.SemaphoreType.DMA,
            pltpu.SemaphoreType.DMA,
            pltpu.SemaphoreType.DMA,
            pltpu.SemaphoreType.DMA,
            pltpu.SemaphoreType.DMA,
            pltpu.SemaphoreType.DMA,
            pltpu.SemaphoreType.DMA,
            pltpu.SemaphoreType.DMA,
        ],
    )
    return call(t_tables, p_tables, q_tables, ei, ei[:2].reshape(
        2, _NT, _WROWS, 128))


def _augment(w2, b2, att_msg, att_self):
    d = _D
    w2a = jnp.concatenate(
        [w2, w2 @ att_msg, jnp.zeros((d, 127), jnp.float32)], axis=1)
    xa = jnp.concatenate(
        [jnp.zeros((d, 129), jnp.float32), att_self,
         jnp.zeros((d, 126), jnp.float32)], axis=1)
    b2a = jnp.concatenate(
        [b2, jnp.dot(b2, att_msg), jnp.zeros((127,), jnp.float32)])
    return w2a, xa, b2a


def _build_edge_index(ci, li):
    cr = ci.reshape(_NT, _EPT // _W, _W)
    lr = li.reshape(_NT, _EPT // _W, _W)
    pad_q = jnp.full((_NT, _PADR, _W), _N, jnp.int32)
    pad_v = (jnp.arange(_NT * _PADR * _W, dtype=jnp.int32) % _N).reshape(
        _NT, _PADR, _W)
    return jnp.stack([
        jnp.concatenate([cr, pad_q], axis=1),
        jnp.concatenate([lr, pad_q], axis=1),
        jnp.concatenate([lr, pad_v], axis=1),
        jnp.concatenate([cr, pad_v], axis=1),
    ])


def kernel(l_size, c_size, l_edge_index, c_edge_index, l_emb, c_emb, params):
    d = _D
    ls = l_emb.shape[0]
    ci = c_edge_index.astype(jnp.int32)
    li = l_edge_index.astype(jnp.int32)
    ei = _build_edge_index(ci, li)

    zpad = jnp.zeros((_SPAD - _N,), jnp.float32)
    zkk = jnp.zeros((d, d), jnp.float32)
    l_embs = [l_emb]
    c_embs = [c_emb]
    for i in range(len(params)):
        p = params[i]
        catt1 = p["c_att_W"][:d, :]
        catt2 = p["c_att_W"][d:, :]
        latt1 = p["l_att_W"][:d, :]
        latt2 = p["l_att_W"][d:, :]

        w2a_l, xa_l, b2a_l = _augment(p["l2c_W2"], p["l2c_b2"], catt2, latt1)
        y_l = _mlp_call(l_emb, p["l2c_W1"], p["l2c_b1"], w2a_l, xa_l, b2a_l)
        w2a_c, xa_c, b2a_c = _augment(p["c2l_W2"], p["c2l_b2"], latt2, catt1)
        y_c = _mlp_call(c_emb, p["c2l_W1"], p["c2l_b1"], w2a_c, xa_c, b2a_c)

        t_tables = jnp.stack([y_l[:, :d], y_c[:, :d]])
        p_tables = jnp.concatenate([y_c[:, 129], zpad, y_c[:, 128], zpad])
        q_tables = jnp.concatenate([y_l[:, 128], zpad, y_l[:, 129], zpad])

        agg = _edge_aggregate(t_tables, p_tables, q_tables, ei)[:, :_N, :]

        r = l_emb.reshape(ls // 2, 2 * d)
        l2l_feat = jnp.concatenate([r[:, d:], r[:, :d]], axis=1).reshape(ls, d)
        l2l_msg = _mlp_call(l2l_feat, p["l2l_W1"], p["l2l_b1"],
                            p["l2l_W2"], zkk, p["l2l_b2"])

        c_emb = _mm_call(jnp.concatenate([c_emb, agg[0]], axis=1),
                         p["c_upd_W"], p["c_upd_b"])
        c_embs.append(c_emb)
        l_emb = _mm_call(jnp.concatenate([l_emb, agg[1], l2l_msg], axis=1),
                         p["l_upd_W"], p["l_upd_b"])
        l_embs.append(l_emb)
    return (tuple(l_embs), tuple(c_embs))


# speedup vs baseline: 1.1442x; 1.1442x over previous
import jax
import jax.numpy as jnp
from jax import lax
from jax.experimental import pallas as pl
from jax.experimental.pallas import tpu as pltpu
from jax.experimental.pallas import tpu_sc as plsc

_D = 128
_N = 10000
_E = 320000
_NT = 16
_EPT = _E // _NT
_W = 16
_ROWS = 1280
_PADR = _ROWS - _EPT // _W
_NBLK = 20
_BR = _ROWS // _NBLK
_NB = 4
_SPAD = 10240
_STRIPE = _SPAD // _NT
_WROWS = _ROWS * _W // 128
_WBR = _WROWS // _NBLK

_PREC = lax.Precision.HIGHEST


def _mlp_body(x_ref, w1_ref, b1_ref, w2_ref, xw_ref, b2_ref, o_ref):
    x = x_ref[...]
    h = jnp.dot(x, w1_ref[...], preferred_element_type=jnp.float32,
                precision=_PREC)
    h = jnp.maximum(h + b1_ref[...], 0.0)
    y = jnp.dot(h, w2_ref[...], preferred_element_type=jnp.float32,
                precision=_PREC)
    y = y + jnp.dot(x, xw_ref[...], preferred_element_type=jnp.float32,
                    precision=_PREC)
    o_ref[...] = y + b2_ref[...]


def _mlp_call(x, w1, b1, w2, xw, b2):
    n, k = x.shape
    f = w2.shape[1]
    br = 1000
    return pl.pallas_call(
        _mlp_body,
        grid=(n // br,),
        in_specs=[
            pl.BlockSpec((br, k), lambda i: (i, 0)),
            pl.BlockSpec((k, k), lambda i: (0, 0)),
            pl.BlockSpec((1, k), lambda i: (0, 0)),
            pl.BlockSpec((k, f), lambda i: (0, 0)),
            pl.BlockSpec((k, f), lambda i: (0, 0)),
            pl.BlockSpec((1, f), lambda i: (0, 0)),
        ],
        out_specs=pl.BlockSpec((br, f), lambda i: (i, 0)),
        out_shape=jax.ShapeDtypeStruct((n, f), jnp.float32),
    )(x, w1, b1.reshape(1, -1), w2, xw, b2.reshape(1, -1))


def _mm_body(x_ref, w_ref, b_ref, o_ref):
    o_ref[...] = jnp.dot(
        x_ref[...], w_ref[...], preferred_element_type=jnp.float32,
        precision=_PREC) + b_ref[...]


def _mm_call(x, w, b):
    n, k = x.shape
    f = w.shape[1]
    br = 1000
    return pl.pallas_call(
        _mm_body,
        grid=(n // br,),
        in_specs=[
            pl.BlockSpec((br, k), lambda i: (i, 0)),
            pl.BlockSpec((k, f), lambda i: (0, 0)),
            pl.BlockSpec((1, f), lambda i: (0, 0)),
        ],
        out_specs=pl.BlockSpec((br, f), lambda i: (i, 0)),
        out_shape=jax.ShapeDtypeStruct((n, f), jnp.float32),
    )(x, w, b.reshape(1, -1))


def _edge_body(t_hbm, p_hbm, q_hbm, ei_hbm, eiw_hbm, out_hbm,
               p_v, q_v, s_v, segw_b, qiw_b, e_blk, zs_v,
               seg_b, qi_b, src_b, rows0, rows1, rows2, rows3, w_v,
               sp_out, sp_s,
               gsem0, gsem1, gsem2, gsem3, ssem0, ssem1, ssem2, ssem3,
               sem_i, sem_s):
    rows_a = rows0
    d = lax.axis_index("c")
    t = lax.axis_index("s")

    z16 = jnp.zeros((16,), jnp.float32)

    pltpu.sync_copy(p_hbm.at[pl.ds(d * _SPAD, _SPAD)], p_v)
    pltpu.sync_copy(q_hbm.at[pl.ds(d * _SPAD, _SPAD)], q_v)

    for k in range(_STRIPE // 16):
        zs_v[pl.ds(k * 16, 16)] = z16
    for r in range(_W):
        for c8 in range(8):
            rows_a[r, pl.ds(c8 * 16, 16)] = z16
    pltpu.sync_copy(zs_v, sp_s.at[pl.ds(t * _STRIPE, _STRIPE)])

    def _zero_out(i, _):
        pltpu.sync_copy(rows_a, sp_out.at[pl.ds(t * _STRIPE + i * _W, _W)])
        return 0
    lax.fori_loop(0, _STRIPE // _W, _zero_out, 0)

    neg = jnp.full((16,), -3.4e38, jnp.float32)

    def _mx(table):
        def step(i, m):
            return jnp.maximum(m, table[pl.ds(i * 16, 16)])
        mv = lax.fori_loop(0, _SPAD // 16, step, neg)
        w_v[pl.ds(0, 16)] = mv
        m = neg
        for k in range(16):
            m = jnp.maximum(
                m, plsc.load_gather(w_v, [jnp.full((16,), k, jnp.int32)]))
        return m

    m_tot = _mx(p_v) + _mx(q_v)
    m_vec = jnp.maximum(m_tot, 0.2 * m_tot)

    def _exp_logit(ci, qi):
        pg = plsc.load_gather(p_v, [ci])
        qg = plsc.load_gather(q_v, [qi])
        x = pg + qg
        xl = jnp.maximum(x, 0.2 * x)
        return jnp.exp(xl - m_vec)

    plsc.subcore_barrier()

    def _p1_block(b, _):
        c1 = pltpu.async_copy(eiw_hbm.at[0, t, pl.ds(b * _WBR, _WBR)],
                              segw_b, sem_i)
        c2 = pltpu.async_copy(eiw_hbm.at[1, t, pl.ds(b * _WBR, _WBR)],
                              qiw_b, sem_i)
        c1.wait()
        c2.wait()

        for j in range(_WBR):
            for k in range(8):
                ci = segw_b[j, pl.ds(k * 16, 16)]
                e_blk[j, pl.ds(k * 16, 16)] = _exp_logit(
                    ci, qiw_b[j, pl.ds(k * 16, 16)])
        for j in range(_WBR):
            pltpu.async_copy(e_blk.at[j], sp_s.at[segw_b.at[j]], sem_s,
                             add=True)
        for j in range(_WBR):
            pltpu.make_async_copy(
                e_blk.at[0], sp_s.at[segw_b.at[0]], sem_s).wait()
        return 0
    lax.fori_loop(0, _NBLK, _p1_block, 0)

    plsc.subcore_barrier()
    pltpu.sync_copy(sp_s, s_v)

    t_view = t_hbm.at[d]
    bufs = ((rows0, gsem0, ssem0), (rows1, gsem1, ssem1),
            (rows2, gsem2, ssem2), (rows3, gsem3, ssem3))

    def _scatter(buf, ssem, jj):
        @pl.when(d == 0)
        def _():
            pltpu.async_copy(buf, sp_out.at[seg_b.at[jj]], ssem, add=True)

        @pl.when(d == 1)
        def _():
            pltpu.async_copy(buf, sp_out.at[qi_b.at[jj]], ssem, add=True)

    def _p2_block(b, _):
        c1 = pltpu.async_copy(ei_hbm.at[0, t, pl.ds(b * _BR, _BR)], seg_b,
                              sem_i)
        c2 = pltpu.async_copy(ei_hbm.at[1, t, pl.ds(b * _BR, _BR)], qi_b,
                              sem_i)
        c3 = pltpu.async_copy(ei_hbm.at[2 + d, t, pl.ds(b * _BR, _BR)],
                              src_b, sem_i)
        c1.wait()
        c2.wait()
        c3.wait()

        for k in range(_NB - 1):
            pltpu.async_copy(t_view.at[src_b.at[k]], bufs[k][0], bufs[k][1])

        def outer(jo, _):
            j0 = jo * _NB
            for bb in range(_NB):
                buf, gsem, ssem = bufs[bb]
                nbuf, ngsem, nssem = bufs[(bb + _NB - 1) % _NB]
                jj = j0 + bb

                @pl.when(jj + _NB - 1 < _BR)
                def _():
                    @pl.when(jj >= 1)
                    def _():
                        pltpu.make_async_copy(
                            nbuf, sp_out.at[seg_b.at[0]], nssem).wait()
                    pltpu.async_copy(
                        t_view.at[src_b.at[jj + _NB - 1]], nbuf, ngsem)

                pltpu.make_async_copy(
                    t_view.at[src_b.at[jj]], buf, gsem).wait()

                ci = seg_b[jj, :]
                e = _exp_logit(ci, qi_b[jj, :])
                sg = plsc.load_gather(s_v, [ci])
                wv = e / (sg + 1e-16)

                gdn = lax.GatherDimensionNumbers(
                    offset_dims=(), collapsed_slice_dims=(0,),
                    start_index_map=(0,))

                def scale_row(r, _):
                    spl = lax.gather(
                        wv, jnp.full((16, 1), r, jnp.int32), gdn, (1,),
                        mode=lax.GatherScatterMode.PROMISE_IN_BOUNDS)
                    for c8 in range(8):
                        buf[r, pl.ds(c8 * 16, 16)] = (
                            buf[r, pl.ds(c8 * 16, 16)] * spl)
                    return 0
                lax.fori_loop(0, _W, scale_row, 0)

                _scatter(buf, ssem, jj)
            return 0
        lax.fori_loop(0, _BR // _NB, outer, 0)

        for k in range(_NB):
            pltpu.make_async_copy(
                bufs[k][0], sp_out.at[seg_b.at[0]], bufs[k][2]).wait()
        return 0
    lax.fori_loop(0, _NBLK, _p2_block, 0)

    plsc.subcore_barrier()
    pltpu.sync_copy(sp_out.at[pl.ds(t * _STRIPE, _STRIPE)],
                    out_hbm.at[d, pl.ds(t * _STRIPE, _STRIPE)])


def _edge_aggregate(t_tables, p_tables, q_tables, ei):
    mesh = plsc.VectorSubcoreMesh(core_axis_name="c", subcore_axis_name="s")
    call = pl.kernel(
        _edge_body,
        out_type=jax.ShapeDtypeStruct((2, _SPAD, _D), jnp.float32),
        mesh=mesh,
        compiler_params=pltpu.CompilerParams(
            needs_layout_passes=False, use_tc_tiling_on_sc=False),
        scratch_types=[
            pltpu.VMEM((_SPAD,), jnp.float32),
            pltpu.VMEM((_SPAD,), jnp.float32),
            pltpu.VMEM((_SPAD,), jnp.float32),
            pltpu.VMEM((_WBR, 128), jnp.int32),
            pltpu.VMEM((_WBR, 128), jnp.int32),
            pltpu.VMEM((_WBR, 128), jnp.float32),
            pltpu.VMEM((_STRIPE,), jnp.float32),
            pltpu.VMEM((_BR, _W), jnp.int32),
            pltpu.VMEM((_BR, _W), jnp.int32),
            pltpu.VMEM((_BR, _W), jnp.int32),
            pltpu.VMEM((_W, _D), jnp.float32),
            pltpu.VMEM((_W, _D), jnp.float32),
            pltpu.VMEM((_W, _D), jnp.float32),
            pltpu.VMEM((_W, _D), jnp.float32),
            pltpu.VMEM((_W,), jnp.float32),
            pltpu.VMEM_SHARED((_SPAD, _D), jnp.float32),
            pltpu.VMEM_SHARED((_SPAD,), jnp.float32),
            pltpu.SemaphoreType.DMA,
            pltpu.SemaphoreType.DMA,
            pltpu.SemaphoreType.DMA,
            pltpu.SemaphoreType.DMA,
            pltpu.SemaphoreType.DMA,
            pltpu.SemaphoreType.DMA,
            pltpu.SemaphoreType.DMA,
            pltpu.SemaphoreType.DMA,
            pltpu.SemaphoreType.DMA,
            pltpu.SemaphoreType.DMA,
        ],
    )
    return call(t_tables, p_tables, q_tables, ei, ei[:2].reshape(
        2, _NT, _WROWS, 128))


def _augment(w2, b2, att_msg, att_self):
    d = _D
    w2a = jnp.concatenate(
        [w2, w2 @ att_msg, jnp.zeros((d, 127), jnp.float32)], axis=1)
    xa = jnp.concatenate(
        [jnp.zeros((d, 129), jnp.float32), att_self,
         jnp.zeros((d, 126), jnp.float32)], axis=1)
    b2a = jnp.concatenate(
        [b2, jnp.dot(b2, att_msg), jnp.zeros((127,), jnp.float32)])
    return w2a, xa, b2a


def _build_edge_index(ci, li):
    cr = ci.reshape(_NT, _EPT // _W, _W)
    lr = li.reshape(_NT, _EPT // _W, _W)
    pad_q = jnp.full((_NT, _PADR, _W), _N, jnp.int32)
    pad_v = (jnp.arange(_NT * _PADR * _W, dtype=jnp.int32) % _N).reshape(
        _NT, _PADR, _W)
    return jnp.stack([
        jnp.concatenate([cr, pad_q], axis=1),
        jnp.concatenate([lr, pad_q], axis=1),
        jnp.concatenate([lr, pad_v], axis=1),
        jnp.concatenate([cr, pad_v], axis=1),
    ])


def kernel(l_size, c_size, l_edge_index, c_edge_index, l_emb, c_emb, params):
    d = _D
    ls = l_emb.shape[0]
    ci = c_edge_index.astype(jnp.int32)
    li = l_edge_index.astype(jnp.int32)
    ei = _build_edge_index(ci, li)

    zpad = jnp.zeros((_SPAD - _N,), jnp.float32)
    zkk = jnp.zeros((d, d), jnp.float32)
    l_embs = [l_emb]
    c_embs = [c_emb]
    for i in range(len(params)):
        p = params[i]
        catt1 = p["c_att_W"][:d, :]
        catt2 = p["c_att_W"][d:, :]
        latt1 = p["l_att_W"][:d, :]
        latt2 = p["l_att_W"][d:, :]

        w2a_l, xa_l, b2a_l = _augment(p["l2c_W2"], p["l2c_b2"], catt2, latt1)
        y_l = _mlp_call(l_emb, p["l2c_W1"], p["l2c_b1"], w2a_l, xa_l, b2a_l)
        w2a_c, xa_c, b2a_c = _augment(p["c2l_W2"], p["c2l_b2"], latt2, catt1)
        y_c = _mlp_call(c_emb, p["c2l_W1"], p["c2l_b1"], w2a_c, xa_c, b2a_c)

        t_tables = jnp.stack([y_l[:, :d], y_c[:, :d]])
        p_tables = jnp.concatenate([y_c[:, 129], zpad, y_c[:, 128], zpad])
        q_tables = jnp.concatenate([y_l[:, 128], zpad, y_l[:, 129], zpad])

        agg = _edge_aggregate(t_tables, p_tables, q_tables, ei)[:, :_N, :]

        r = l_emb.reshape(ls // 2, 2 * d)
        l2l_feat = jnp.concatenate([r[:, d:], r[:, :d]], axis=1).reshape(ls, d)
        l2l_msg = _mlp_call(l2l_feat, p["l2l_W1"], p["l2l_b1"],
                            p["l2l_W2"], zkk, p["l2l_b2"])

        c_emb = _mm_call(jnp.concatenate([c_emb, agg[0]], axis=1),
                         p["c_upd_W"], p["c_upd_b"])
        c_embs.append(c_emb)
        l_emb = _mm_call(jnp.concatenate([l_emb, agg[1], l2l_msg], axis=1),
                         p["l_upd_W"], p["l_upd_b"])
        l_embs.append(l_emb)
    return (tuple(l_embs), tuple(c_embs))
